# baseline (speedup 1.0000x reference)
import jax
import jax.numpy as jnp
from jax import lax
from jax.experimental import pallas as pl
from jax.experimental.pallas import tpu as pltpu
from jax.experimental.pallas import tpu_sc as plsc

C = 768
HW = 112 * 112
TOPK = 384
L = 16
NSUB = 16
CPT = C // NSUB
TPT = CPT // L
NVREG = C // L


def _mean_body(x_ref, o_ref):
    o_ref[...] = jnp.sum(x_ref[...], axis=1, keepdims=True) * (1.0 / HW)


def _channel_means(x2):
    return pl.pallas_call(
        _mean_body,
        grid=(12,),
        in_specs=[pl.BlockSpec((64, HW), lambda i: (i, 0))],
        out_specs=pl.BlockSpec((64, 1), lambda i: (i, 0)),
        out_shape=jax.ShapeDtypeStruct((C, 1), jnp.float32),
        compiler_params=pltpu.CompilerParams(allow_input_fusion=[True]),
    )(x2)


def _topk_body(means_hbm, out_hbm, means_v, ranks_mine, ids_mine):
    sid = lax.axis_index("s")
    iota = jnp.arange(L, dtype=jnp.int32)
    ones = jnp.ones((L,), jnp.int32)
    zeros = jnp.zeros((L,), jnp.int32)

    pltpu.sync_copy(means_hbm, means_v)

    tb = [(sid * TPT + tl) * L for tl in range(TPT)]
    vs = [means_v[pl.ds(tb[tl], L)] for tl in range(TPT)]
    idx = [iota + tb[tl] for tl in range(TPT)]

    def m_step(m, accs):
        u = means_v[pl.ds(m * L, L)]
        accs = list(accs)
        for k in range(L):
            us = u.at[jnp.full((L,), k, jnp.int32)].get(
                mode='promise_in_bounds')
            j = jnp.full((L,), m * L + k, jnp.int32)
            for tl in range(TPT):
                cond = (us > vs[tl]) | ((us == vs[tl]) & (j > idx[tl]))
                accs[tl] = accs[tl] + jnp.where(cond, ones, zeros)
        return tuple(accs)

    accs = lax.fori_loop(0, NVREG, m_step, (zeros,) * TPT)
    for tl in range(TPT):
        ranks_mine[pl.ds(tl * L, L)] = accs[tl]
        ids_mine[pl.ds(tl * L, L)] = idx[tl]

    pltpu.sync_copy(ids_mine, out_hbm.at[ranks_mine])


def _topk_sc(means):
    mesh = plsc.VectorSubcoreMesh(
        core_axis_name="c", subcore_axis_name="s", num_cores=1)
    f = pl.kernel(
        _topk_body,
        out_type=jax.ShapeDtypeStruct((C,), jnp.int32),
        mesh=mesh,
        scratch_types=[
            pltpu.VMEM((C,), jnp.float32),
            pltpu.VMEM((CPT,), jnp.int32),
            pltpu.VMEM((CPT,), jnp.int32),
        ],
    )
    return f(means)


@jax.jit
def kernel(input):
    means = _channel_means(input.reshape(C, HW)).reshape(C)
    return _topk_sc(means)[:TOPK]


# speedup vs baseline: 1.1866x; 1.1866x over previous
import jax
import jax.numpy as jnp
from jax import lax
from jax.experimental import pallas as pl
from jax.experimental.pallas import tpu as pltpu
from jax.experimental.pallas import tpu_sc as plsc

C = 768
HW = 112 * 112
TOPK = 384
L = 16
NSUB = 16
CPT = C // NSUB
TPT = CPT // L
NVREG = C // L


def _mean_body(x_ref, o_ref):
    o_ref[...] = jnp.sum(x_ref[...], axis=1, keepdims=True) * (1.0 / HW)


def _channel_means(x2):
    return pl.pallas_call(
        _mean_body,
        grid=(12,),
        in_specs=[pl.BlockSpec((64, HW), lambda i: (i, 0))],
        out_specs=pl.BlockSpec((64, 1), lambda i: (i, 0)),
        out_shape=jax.ShapeDtypeStruct((C, 1), jnp.float32),
        compiler_params=pltpu.CompilerParams(allow_input_fusion=[True]),
    )(x2)


def _topk_body(means_hbm, out_hbm, means_v, ranks_mine, ids_mine, out_sh,
               out_v):
    sid = lax.axis_index("s")
    iota = jnp.arange(L, dtype=jnp.int32)
    ones = jnp.ones((L,), jnp.int32)
    zeros = jnp.zeros((L,), jnp.int32)

    pltpu.sync_copy(means_hbm, means_v)

    tb = [(sid * TPT + tl) * L for tl in range(TPT)]
    vs = [means_v[pl.ds(tb[tl], L)] for tl in range(TPT)]
    idx = [iota + tb[tl] for tl in range(TPT)]

    def m_step(m, accs):
        u = means_v[pl.ds(m * L, L)]
        accs = list(accs)
        for k in range(L):
            us = u.at[jnp.full((L,), k, jnp.int32)].get(
                mode='promise_in_bounds')
            j = jnp.full((L,), m * L + k, jnp.int32)
            for tl in range(TPT):
                cond = (us > vs[tl]) | ((us == vs[tl]) & (j > idx[tl]))
                accs[tl] = accs[tl] + jnp.where(cond, ones, zeros)
        return tuple(accs)

    accs = lax.fori_loop(0, NVREG, m_step, (zeros,) * TPT)
    for tl in range(TPT):
        ranks_mine[pl.ds(tl * L, L)] = accs[tl]
        ids_mine[pl.ds(tl * L, L)] = idx[tl]

    pltpu.sync_copy(ids_mine, out_sh.at[ranks_mine])
    plsc.subcore_barrier()

    @pl.when(sid == 0)
    def _():
        pltpu.sync_copy(out_sh.at[pl.ds(0, TOPK)], out_v)
        pltpu.sync_copy(out_v, out_hbm)


def _topk_sc(means):
    mesh = plsc.VectorSubcoreMesh(
        core_axis_name="c", subcore_axis_name="s", num_cores=1)
    f = pl.kernel(
        _topk_body,
        out_type=jax.ShapeDtypeStruct((TOPK,), jnp.int32),
        mesh=mesh,
        scratch_types=[
            pltpu.VMEM((C,), jnp.float32),
            pltpu.VMEM((CPT,), jnp.int32),
            pltpu.VMEM((CPT,), jnp.int32),
            pltpu.VMEM_SHARED((C,), jnp.int32),
            pltpu.VMEM((TOPK,), jnp.int32),
        ],
    )
    return f(means)


@jax.jit
def kernel(input):
    means = _channel_means(input.reshape(C, HW)).reshape(C)
    return _topk_sc(means)
